# baseline (speedup 1.0000x reference)
import jax
import jax.numpy as jnp
from jax.experimental import pallas as pl

_HI = jax.lax.Precision.HIGHEST


def _split3(v):
    p1 = v.astype(jnp.bfloat16).astype(jnp.float32)
    r1 = v - p1
    p2 = r1.astype(jnp.bfloat16).astype(jnp.float32)
    p3 = (r1 - p2).astype(jnp.bfloat16).astype(jnp.float32)
    return p1, p2, p3


def _prep_body(emb_ref, o_ref):
    E = emb_ref[...]
    ones_col = jnp.ones((E.shape[1], 1), jnp.float32)
    e2 = jax.lax.dot_general(E * E, ones_col, (((1,), (0,)), ((), ())),
                             precision=_HI, preferred_element_type=jnp.float32)
    p1, p2, p3 = _split3(e2)
    ones3 = jnp.ones((E.shape[0], 3), jnp.float32)
    o_ref[...] = jnp.concatenate([2.0 * E, ones3, -p1, -p2, -p3], axis=1)


def _vq_chunk(X, Wi, b_in, E_bf, Wo_bf, eaug):
    Tc = X.shape[1]

    Z = jax.lax.dot_general(X, Wi, (((0,), (0,)), ((), ())),
                            preferred_element_type=jnp.float32)
    Z = Z + b_in

    x2 = jnp.sum(Z * Z, axis=1, keepdims=True)
    q1, q2, q3 = _split3(x2)
    ones3 = jnp.ones((Tc, 3), jnp.float32)
    Zaug = jnp.concatenate([Z, -q1, -q2, -q3, ones3], axis=1)

    dist = jax.lax.dot_general(Zaug, eaug, (((1,), (1,)), ((), ())),
                               preferred_element_type=jnp.float32)

    K = dist.shape[1]
    M = jnp.max(dist, axis=1, keepdims=True)
    iota = jax.lax.broadcasted_iota(jnp.int32, dist.shape, 1)
    cand = jnp.where(dist == M, iota, jnp.int32(K))
    ind = jnp.min(cand, axis=1)
    oh = (iota == ind[:, None]).astype(jnp.bfloat16)

    Q = jax.lax.dot_general(E_bf, oh, (((0,), (1,)), ((), ())),
                            preferred_element_type=jnp.float32)
    Y = jax.lax.dot_general(Wo_bf, Q.astype(jnp.bfloat16),
                            (((0,), (0,)), ((), ())),
                            preferred_element_type=jnp.float32)
    return Y


_N_CHUNKS = 2


def _vq_body(x_ref, w_in_ref, b_in_ref, emb_ref, w_out_ref, b_out_ref,
             eaug_ref, o_ref):
    Wi = w_in_ref[...]
    b_in = b_in_ref[...]
    E_bf = emb_ref[...].astype(jnp.bfloat16)
    Wo_bf = w_out_ref[...].astype(jnp.bfloat16)
    eaug = eaug_ref[...]
    b_out = b_out_ref[...]

    Tb = x_ref.shape[2]
    Tc = Tb // _N_CHUNKS
    for h in range(_N_CHUNKS):
        X = x_ref[0, :, h * Tc:(h + 1) * Tc]
        Y = _vq_chunk(X, Wi, b_in, E_bf, Wo_bf, eaug)
        o_ref[0, :, h * Tc:(h + 1) * Tc] = Y + b_out


def kernel(hidden_states, W_in, b_in, embed, W_out, b_out):
    B, D, T = hidden_states.shape
    K, CD = embed.shape
    Tb = 1024

    b_in2 = b_in.reshape(1, CD)
    b_out2 = b_out.reshape(D, 1)

    eaug = pl.pallas_call(
        _prep_body,
        out_shape=jax.ShapeDtypeStruct((K, CD + 6), jnp.float32),
    )(embed)

    grid = (B, T // Tb)
    out = pl.pallas_call(
        _vq_body,
        grid=grid,
        in_specs=[
            pl.BlockSpec((1, D, Tb), lambda b, t: (b, 0, t)),
            pl.BlockSpec((D, CD), lambda b, t: (0, 0)),
            pl.BlockSpec((1, CD), lambda b, t: (0, 0)),
            pl.BlockSpec((K, CD), lambda b, t: (0, 0)),
            pl.BlockSpec((CD, D), lambda b, t: (0, 0)),
            pl.BlockSpec((D, 1), lambda b, t: (0, 0)),
            pl.BlockSpec((K, CD + 6), lambda b, t: (0, 0)),
        ],
        out_specs=pl.BlockSpec((1, D, Tb), lambda b, t: (b, 0, t)),
        out_shape=jax.ShapeDtypeStruct((B, D, T), jnp.float32),
    )(hidden_states, W_in, b_in2, embed, W_out, b_out2, eaug)
    return out


# speedup vs baseline: 1.1730x; 1.1730x over previous
import jax
import jax.numpy as jnp
from jax.experimental import pallas as pl

_HI = jax.lax.Precision.HIGHEST


def _e2_body(emb_ref, o_ref):
    E = emb_ref[...]
    ones = jnp.ones((8, E.shape[1]), jnp.float32)
    o_ref[...] = jax.lax.dot_general(
        ones, E * E, (((1,), (1,)), ((), ())),
        precision=_HI, preferred_element_type=jnp.float32)


def _vq_body(x_ref, w_in_ref, b_in_ref, emb_ref, w_out_ref, b_out_ref,
             e2_ref, o_ref):
    X = x_ref[0]
    Wi = w_in_ref[...]
    E = emb_ref[...]
    Wo = w_out_ref[...]

    Z = jax.lax.dot_general(X, Wi, (((0,), (0,)), ((), ())),
                            preferred_element_type=jnp.float32)
    Z = Z + b_in_ref[...]

    x2 = jnp.sum(Z * Z, axis=1, keepdims=True)
    S = jax.lax.dot_general(Z, E, (((1,), (1,)), ((), ())),
                            preferred_element_type=jnp.float32)
    dist = -(x2 - 2.0 * S + e2_ref[...])

    ind = jnp.argmax(dist, axis=1)
    iota = jax.lax.broadcasted_iota(jnp.int32, dist.shape, 1)
    oh = (iota == ind[:, None]).astype(jnp.float32)

    Q = jax.lax.dot_general(E, oh, (((0,), (1,)), ((), ())),
                            preferred_element_type=jnp.float32)
    Y = jax.lax.dot_general(Wo, Q, (((0,), (0,)), ((), ())),
                            preferred_element_type=jnp.float32)
    o_ref[0] = Y + b_out_ref[...]


def kernel(hidden_states, W_in, b_in, embed, W_out, b_out):
    B, D, T = hidden_states.shape
    K, CD = embed.shape
    Tb = 1024

    b_in2 = b_in.reshape(1, CD)
    b_out2 = b_out.reshape(D, 1)

    e2 = pl.pallas_call(
        _e2_body,
        out_shape=jax.ShapeDtypeStruct((8, K), jnp.float32),
    )(embed)[0:1]

    grid = (B, T // Tb)
    out = pl.pallas_call(
        _vq_body,
        grid=grid,
        in_specs=[
            pl.BlockSpec((1, D, Tb), lambda b, t: (b, 0, t)),
            pl.BlockSpec((D, CD), lambda b, t: (0, 0)),
            pl.BlockSpec((1, CD), lambda b, t: (0, 0)),
            pl.BlockSpec((K, CD), lambda b, t: (0, 0)),
            pl.BlockSpec((CD, D), lambda b, t: (0, 0)),
            pl.BlockSpec((D, 1), lambda b, t: (0, 0)),
            pl.BlockSpec((1, K), lambda b, t: (0, 0)),
        ],
        out_specs=pl.BlockSpec((1, D, Tb), lambda b, t: (b, 0, t)),
        out_shape=jax.ShapeDtypeStruct((B, D, T), jnp.float32),
    )(hidden_states, W_in, b_in2, embed, W_out, b_out2, e2)
    return out


# speedup vs baseline: 1.2789x; 1.0903x over previous
import jax
import jax.numpy as jnp
from jax.experimental import pallas as pl

_HI = jax.lax.Precision.HIGHEST


def _e2_body(emb_ref, o_ref):
    E = emb_ref[...]
    ones = jnp.ones((8, E.shape[1]), jnp.float32)
    o_ref[...] = jax.lax.dot_general(
        ones, E * E, (((1,), (1,)), ((), ())),
        precision=_HI, preferred_element_type=jnp.float32)


def _vq_body(x_ref, w_in_ref, b_in_ref, emb_ref, w_out_ref, b_out_ref,
             e2_ref, o_ref):
    X = x_ref[0]
    Wi = w_in_ref[...]
    E = emb_ref[...]
    Wo = w_out_ref[...]

    Z = jax.lax.dot_general(X, Wi, (((0,), (0,)), ((), ())),
                            preferred_element_type=jnp.float32)
    Z = Z + b_in_ref[...]

    x2 = jnp.sum(Z * Z, axis=1, keepdims=True)
    S = jax.lax.dot_general(Z, E, (((1,), (1,)), ((), ())),
                            preferred_element_type=jnp.float32)
    dist = -(x2 - 2.0 * S + e2_ref[...])

    ind = jnp.argmax(dist, axis=1)
    iota = jax.lax.broadcasted_iota(jnp.int32, dist.shape, 1)
    oh = (iota == ind[:, None]).astype(jnp.float32)

    Q = jax.lax.dot_general(E, oh, (((0,), (1,)), ((), ())),
                            preferred_element_type=jnp.float32)
    Y = jax.lax.dot_general(Wo, Q, (((0,), (0,)), ((), ())),
                            preferred_element_type=jnp.float32)
    o_ref[0] = Y + b_out_ref[...]


def kernel(hidden_states, W_in, b_in, embed, W_out, b_out):
    B, D, T = hidden_states.shape
    K, CD = embed.shape
    Tb = 2048

    b_in2 = b_in.reshape(1, CD)
    b_out2 = b_out.reshape(D, 1)

    e2 = pl.pallas_call(
        _e2_body,
        out_shape=jax.ShapeDtypeStruct((8, K), jnp.float32),
    )(embed)[0:1]

    grid = (B, T // Tb)
    out = pl.pallas_call(
        _vq_body,
        grid=grid,
        in_specs=[
            pl.BlockSpec((1, D, Tb), lambda b, t: (b, 0, t)),
            pl.BlockSpec((D, CD), lambda b, t: (0, 0)),
            pl.BlockSpec((1, CD), lambda b, t: (0, 0)),
            pl.BlockSpec((K, CD), lambda b, t: (0, 0)),
            pl.BlockSpec((CD, D), lambda b, t: (0, 0)),
            pl.BlockSpec((D, 1), lambda b, t: (0, 0)),
            pl.BlockSpec((1, K), lambda b, t: (0, 0)),
        ],
        out_specs=pl.BlockSpec((1, D, Tb), lambda b, t: (b, 0, t)),
        out_shape=jax.ShapeDtypeStruct((B, D, T), jnp.float32),
    )(hidden_states, W_in, b_in2, embed, W_out, b_out2, e2)
    return out
